# baseline (speedup 1.0000x reference)
import functools

import jax
import jax.numpy as jnp
from jax import lax
from jax.experimental import pallas as pl
from jax.experimental.pallas import tpu as pltpu
from jax.experimental.pallas import tpu_sc as plsc

NC = 2
NS = 16
NW = NC * NS
CHUNK = 128


def _sc_gather(u_idx, i_idx, j_idx, U_latent, I_latent, U_visual,
               visual_features, item_bias):
    B = u_idx.shape[0]
    K = U_latent.shape[1]
    F = visual_features.shape[1]
    bpw = B // NW
    nch = bpw // CHUNK

    mesh = plsc.VectorSubcoreMesh(core_axis_name="c", subcore_axis_name="s")

    out_type = (
        jax.ShapeDtypeStruct((B, K), jnp.float32),
        jax.ShapeDtypeStruct((B, K), jnp.float32),
        jax.ShapeDtypeStruct((B, K), jnp.float32),
        jax.ShapeDtypeStruct((B, K), jnp.float32),
        jax.ShapeDtypeStruct((B, F), jnp.float32),
        jax.ShapeDtypeStruct((B, F), jnp.float32),
        jax.ShapeDtypeStruct((B,), jnp.float32),
        jax.ShapeDtypeStruct((B,), jnp.float32),
    )

    @functools.partial(
        pl.kernel,
        out_type=out_type,
        mesh=mesh,
        scratch_types=[
            pltpu.VMEM((CHUNK,), jnp.int32),
            pltpu.VMEM((CHUNK,), jnp.int32),
            pltpu.VMEM((CHUNK,), jnp.int32),
            pltpu.VMEM((CHUNK, K), jnp.float32),
            pltpu.VMEM((CHUNK, K), jnp.float32),
            pltpu.VMEM((CHUNK, K), jnp.float32),
            pltpu.VMEM((CHUNK, K), jnp.float32),
            pltpu.VMEM((CHUNK, F), jnp.float32),
            pltpu.VMEM((CHUNK, F), jnp.float32),
            pltpu.VMEM((CHUNK,), jnp.float32),
            pltpu.VMEM((CHUNK,), jnp.float32),
            pltpu.SemaphoreType.DMA,
        ],
        compiler_params=pltpu.CompilerParams(use_tc_tiling_on_sc=False),
    )
    def k(u_hbm, i_hbm, j_hbm, UL, IL, UV, VF, IB,
          o_ul, o_ii, o_ij, o_uv, o_vi, o_vj, o_bi, o_bj,
          u_c, i_c, j_c, bul, bii, bij, buv, bvi, bvj, bbi, bbj, sem):
        cid = lax.axis_index("c")
        sid = lax.axis_index("s")
        wid = sid * NC + cid
        base = wid * bpw
        for c in range(nch):
            off = base + c * CHUNK
            sl = pl.ds(off, CHUNK)
            pltpu.sync_copy(u_hbm.at[sl], u_c)
            pltpu.sync_copy(i_hbm.at[sl], i_c)
            pltpu.sync_copy(j_hbm.at[sl], j_c)
            cps = [
                pltpu.async_copy(UL.at[u_c], bul, sem),
                pltpu.async_copy(IL.at[i_c], bii, sem),
                pltpu.async_copy(IL.at[j_c], bij, sem),
                pltpu.async_copy(UV.at[u_c], buv, sem),
                pltpu.async_copy(VF.at[i_c], bvi, sem),
                pltpu.async_copy(VF.at[j_c], bvj, sem),
                pltpu.async_copy(IB.at[i_c], bbi, sem),
                pltpu.async_copy(IB.at[j_c], bbj, sem),
            ]
            for cp in cps:
                cp.wait()
            pltpu.sync_copy(bul, o_ul.at[sl])
            pltpu.sync_copy(bii, o_ii.at[sl])
            pltpu.sync_copy(bij, o_ij.at[sl])
            pltpu.sync_copy(buv, o_uv.at[sl])
            pltpu.sync_copy(bvi, o_vi.at[sl])
            pltpu.sync_copy(bvj, o_vj.at[sl])
            pltpu.sync_copy(bbi, o_bi.at[sl])
            pltpu.sync_copy(bbj, o_bj.at[sl])

    return k(u_idx, i_idx, j_idx, U_latent, I_latent, U_visual,
             visual_features, item_bias)


def _tc_compute(ulu, ili, ilj, uvu, vfi, vfj, ibi, ibj, W_proj, beta):
    B, K = ulu.shape
    F = vfi.shape[1]
    BLK = 1024
    NB = B // BLK
    ibi3 = ibi.reshape(NB, 1, BLK)
    ibj3 = ibj.reshape(NB, 1, BLK)

    def body(ulu_r, ili_r, ilj_r, uvu_r, vfi_r, vfj_r, ibi_r, ibj_r,
             W_r, beta_r, o_r):
        dvf = vfi_r[...] - vfj_r[...]
        dil = ili_r[...] - ilj_r[...]
        proj = lax.dot_general(dvf, W_r[...], (((1,), (1,)), ((), ())),
                               preferred_element_type=jnp.float32)
        lat = jnp.sum(ulu_r[...] * dil, axis=1)
        vis = jnp.sum(uvu_r[...] * proj, axis=1)
        bet = jnp.sum(dvf * beta_r[...], axis=1)
        o_r[0, 0, :] = ibi_r[0, 0, :] - ibj_r[0, 0, :] + lat + vis + bet

    bk = pl.BlockSpec((BLK, K), lambda b: (b, 0))
    bf = pl.BlockSpec((BLK, F), lambda b: (b, 0))
    bs = pl.BlockSpec((1, 1, BLK), lambda b: (b, 0, 0))
    out3 = pl.pallas_call(
        body,
        grid=(NB,),
        in_specs=[bk, bk, bk, bk, bf, bf, bs, bs,
                  pl.BlockSpec((K, F), lambda b: (0, 0)),
                  pl.BlockSpec((1, F), lambda b: (0, 0))],
        out_specs=bs,
        out_shape=jax.ShapeDtypeStruct((NB, 1, BLK), jnp.float32),
    )(ulu, ili, ilj, uvu, vfi, vfj, ibi3, ibj3, W_proj, beta)
    return out3.reshape(B)


def kernel(trg_batch, U_latent, I_latent, U_visual, W_proj, b_proj,
           beta_dash, user_bias, item_bias, visual_features):
    u_idx = trg_batch[:, 0].astype(jnp.int32)
    i_idx = trg_batch[:, 1].astype(jnp.int32)
    j_idx = trg_batch[:, 2].astype(jnp.int32)
    gathered = _sc_gather(u_idx, i_idx, j_idx, U_latent, I_latent,
                          U_visual, visual_features, item_bias)
    ulu, ili, ilj, uvu, vfi, vfj, ibi, ibj = gathered
    return _tc_compute(ulu, ili, ilj, uvu, vfi, vfj, ibi, ibj,
                       W_proj, beta_dash)
